# baseline (speedup 1.0000x reference)
import jax
import jax.numpy as jnp
from jax.experimental import pallas as pl
from jax.experimental.pallas import tpu as pltpu

_NUM_ROWS = 100000
_ROW = 30
_LBLK = 128
_OUT_R = 6
_OUT_C = 5


def _round_idx(x0):
    y = x0 * jnp.float32(_NUM_ROWS - 1)
    n = y.astype(jnp.int32)
    f = y - n.astype(jnp.float32)
    half = jnp.float32(0.5)
    up = jnp.logical_or(f > half,
                        jnp.logical_and(f == half, (n & 1) == 1))
    return n + up.astype(jnp.int32)


def _lookup_body(x_ref, tblk, out_vmem):
    m = _round_idx(x_ref[0]) % _LBLK
    lane = jax.lax.broadcasted_iota(jnp.int32, (_ROW, _LBLK), 1)
    sel = jnp.where(lane == m, tblk[...], jnp.float32(0.0))
    col = jnp.sum(sel, axis=1, keepdims=True)
    out_vmem[...] = col.reshape(_OUT_R, _OUT_C)


@jax.jit
def _lookup(x, table):
    grid_spec = pltpu.PrefetchScalarGridSpec(
        num_scalar_prefetch=1,
        grid=(1,),
        in_specs=[
            pl.BlockSpec((_ROW, _LBLK),
                         lambda g, x_ref: (0, _round_idx(x_ref[0]) // _LBLK)),
        ],
        out_specs=pl.BlockSpec((_OUT_R, _OUT_C), lambda g, x_ref: (0, 0)),
    )
    return pl.pallas_call(
        _lookup_body,
        grid_spec=grid_spec,
        out_shape=jax.ShapeDtypeStruct((_OUT_R, _OUT_C), jnp.float32),
        compiler_params=pltpu.CompilerParams(
            vmem_limit_bytes=256 * 1024,
            disable_bounds_checks=True,
        ),
    )(x.reshape(1),
      pltpu.with_memory_space_constraint(table.T, pltpu.MemorySpace.HBM))


def kernel(x, table):
    return _lookup(x, table)


# speedup vs baseline: 1.0477x; 1.0477x over previous
import jax
import jax.numpy as jnp
from jax.experimental import pallas as pl
from jax.experimental.pallas import tpu as pltpu

_NUM_ROWS = 100000
_ROW = 30
_LBLK = 128
_OUT_R = 6
_OUT_C = 5


def _round_idx(x0):
    y = x0 * jnp.float32(_NUM_ROWS - 1)
    n = y.astype(jnp.int32)
    f = y - n.astype(jnp.float32)
    half = jnp.float32(0.5)
    up = jnp.logical_or(f > half,
                        jnp.logical_and(f == half, (n & 1) == 1))
    return n + up.astype(jnp.int32)


def _lookup_body(x_ref, tblk, out_vmem):
    m = _round_idx(x_ref[0, 0]) % _LBLK
    lane = jax.lax.broadcasted_iota(jnp.int32, (_ROW, _LBLK), 1)
    sel = jnp.where(lane == m, tblk[...], jnp.float32(0.0))
    col = jnp.sum(sel, axis=1, keepdims=True)
    out_vmem[...] = col.reshape(_OUT_R, _OUT_C)


@jax.jit
def _lookup(x, table):
    grid_spec = pltpu.PrefetchScalarGridSpec(
        num_scalar_prefetch=1,
        grid=(1,),
        in_specs=[
            pl.BlockSpec((_ROW, _LBLK),
                         lambda g, x_ref: (0, _round_idx(x_ref[0, 0]) // _LBLK)),
        ],
        out_specs=pl.BlockSpec((_OUT_R, _OUT_C), lambda g, x_ref: (0, 0)),
    )
    return pl.pallas_call(
        _lookup_body,
        grid_spec=grid_spec,
        out_shape=jax.ShapeDtypeStruct((_OUT_R, _OUT_C), jnp.float32),
        compiler_params=pltpu.CompilerParams(
            vmem_limit_bytes=256 * 1024,
            disable_bounds_checks=True,
        ),
    )(x,
      pltpu.with_memory_space_constraint(table.T, pltpu.MemorySpace.HBM))


def kernel(x, table):
    return _lookup(x, table)
